# baseline (speedup 1.0000x reference)
import jax
import jax.numpy as jnp
from jax.experimental import pallas as pl
from jax.experimental.pallas import tpu as pltpu

BATCH = 16384
EMBED = 64
_UNROLL = 8


def _body(uid_ref, iid_ref, ut_ref, it_ref, w1u_ref, w1i_ref, b1_ref,
          w2_ref, b2_ref, out_ref, urows, irows, sem):
    def issue(g, _):
        base = g * _UNROLL
        for j in range(_UNROLL):
            u = uid_ref[base + j]
            pltpu.make_async_copy(
                ut_ref.at[pl.ds(u, 1)], urows.at[pl.ds(base + j, 1)], sem
            ).start()
            v = iid_ref[base + j]
            pltpu.make_async_copy(
                it_ref.at[pl.ds(v, 1)], irows.at[pl.ds(base + j, 1)], sem
            ).start()
        return 0

    jax.lax.fori_loop(0, BATCH // _UNROLL, issue, 0)
    pltpu.make_async_copy(ut_ref.at[pl.ds(0, BATCH)], urows, sem).wait()
    pltpu.make_async_copy(it_ref.at[pl.ds(0, BATCH)], irows, sem).wait()

    h = (jnp.dot(urows[...], w1u_ref[...], preferred_element_type=jnp.float32)
         + jnp.dot(irows[...], w1i_ref[...], preferred_element_type=jnp.float32)
         + b1_ref[...])
    h = jax.nn.sigmoid(h)
    out_ref[...] = (jnp.sum(h * w2_ref[...], axis=1, keepdims=True)
                    + b2_ref[...])


def kernel(user_ids, item_ids, user_table, item_table, W1, b1, W2, b2):
    w1u = W1[:EMBED]
    w1i = W1[EMBED:]
    b1r = b1.reshape(1, EMBED)
    w2r = W2.reshape(1, EMBED)
    b2r = b2.reshape(1, 1)
    return pl.pallas_call(
        _body,
        in_specs=[
            pl.BlockSpec(memory_space=pltpu.SMEM),
            pl.BlockSpec(memory_space=pltpu.SMEM),
            pl.BlockSpec(memory_space=pl.ANY),
            pl.BlockSpec(memory_space=pl.ANY),
            pl.BlockSpec(memory_space=pltpu.VMEM),
            pl.BlockSpec(memory_space=pltpu.VMEM),
            pl.BlockSpec(memory_space=pltpu.VMEM),
            pl.BlockSpec(memory_space=pltpu.VMEM),
            pl.BlockSpec(memory_space=pltpu.VMEM),
        ],
        out_specs=pl.BlockSpec(memory_space=pltpu.VMEM),
        out_shape=jax.ShapeDtypeStruct((BATCH, 1), jnp.float32),
        scratch_shapes=[
            pltpu.VMEM((BATCH, EMBED), jnp.float32),
            pltpu.VMEM((BATCH, EMBED), jnp.float32),
            pltpu.SemaphoreType.DMA,
        ],
    )(user_ids, item_ids, user_table, item_table, w1u, w1i, b1r, w2r, b2r)
